# baseline (speedup 1.0000x reference)
import jax
import jax.numpy as jnp
from jax import lax
from jax.experimental import pallas as pl
from jax.experimental.pallas import tpu as pltpu, tpu_sc as plsc

E = 1_600_000
D = 16
OUT_D = 46
L = 16
C = 512
NW = 32
N_CHUNKS = E // C
CPW = -(-N_CHUNKS // NW)
GRPS = C // L

_DNUMS = lax.GatherDimensionNumbers(
    offset_dims=(), collapsed_slice_dims=(0,), start_index_map=(0,))


def _lut16(table_row, idx):
    return lax.gather(table_row, idx.reshape(L, 1), _DNUMS, (1,),
                      mode=lax.GatherScatterMode.PROMISE_IN_BOUNDS)


def _sc_body(ea_hbm, tc_hbm, out_hbm,
             tc_v, in0, in1, st0, st1, sin0, sin1, sout0, sout1):
    wid = lax.axis_index("s") * 2 + lax.axis_index("c")
    pltpu.sync_copy(tc_hbm, tc_v)

    ins = (sin0, sin1)
    outs = (sout0, sout1)
    inbufs = (in0, in1)
    stages = (st0, st1)

    def in_copy(g, b):
        return pltpu.make_async_copy(
            ea_hbm.at[:, pl.ds(g * C, C)], inbufs[b], ins[b])

    def out_copy(g, b):
        return pltpu.make_async_copy(
            stages[b], out_hbm.at[:, pl.ds(g * C, C)], outs[b])

    def compute(b):
        ib = inbufs[b]
        sb = stages[b]

        @plsc.parallel_loop(0, GRPS, step=1, unroll=4)
        def grp(i):
            goff = i * L
            ti = ib[0, pl.ds(goff, L)].astype(jnp.int32)
            ri = ib[1, pl.ds(goff, L)].astype(jnp.int32)
            for j in range(D):
                sb[j, pl.ds(goff, L)] = _lut16(tc_v[j, :], ti)
            for j in range(D):
                sb[D + j, pl.ds(goff, L)] = _lut16(tc_v[D + j, :], ri)
            for c in range(2, D):
                sb[30 + c, pl.ds(goff, L)] = ib[c, pl.ds(goff, L)]

    def do_chunk(k, b):
        g = k * NW + wid

        @pl.when(g < N_CHUNKS)
        def _():
            in_copy(g, b).wait()

            @pl.when(k >= 2)
            def _():
                out_copy(g, b).wait()

            compute(b)
            out_copy(g, b).start()

            @pl.when(g + 2 * NW < N_CHUNKS)
            def _():
                in_copy(g + 2 * NW, b).start()

    @pl.when(wid < N_CHUNKS)
    def _():
        in_copy(wid, 0).start()

    @pl.when(NW + wid < N_CHUNKS)
    def _():
        in_copy(NW + wid, 1).start()

    def pair_body(i, carry):
        do_chunk(2 * i, 0)
        do_chunk(2 * i + 1, 1)
        return carry

    lax.fori_loop(0, CPW // 2, pair_body, 0)

    out_copy(wid, 0).wait()
    out_copy(wid, 1).wait()


@jax.jit
def _run(ea_t, tcols):
    mesh = plsc.VectorSubcoreMesh(core_axis_name="c", subcore_axis_name="s")
    f = pl.kernel(
        _sc_body,
        out_type=jax.ShapeDtypeStruct((OUT_D, E), jnp.float32),
        mesh=mesh,
        scratch_types=[
            pltpu.VMEM((2 * D, L), jnp.float32),
            pltpu.VMEM((D, C), jnp.float32),
            pltpu.VMEM((D, C), jnp.float32),
            pltpu.VMEM((OUT_D, C), jnp.float32),
            pltpu.VMEM((OUT_D, C), jnp.float32),
            pltpu.SemaphoreType.DMA,
            pltpu.SemaphoreType.DMA,
            pltpu.SemaphoreType.DMA,
            pltpu.SemaphoreType.DMA,
        ],
    )
    return f(ea_t, tcols)


def kernel(edge_attr, W_type, W_ring):
    tcols = jnp.concatenate([
        jnp.pad(W_type.T, ((0, 0), (0, L - 10))),
        jnp.pad(W_ring.T, ((0, 0), (0, L - 10))),
    ], axis=0)
    return _run(edge_attr.T, tcols).T


# speedup vs baseline: 1.0777x; 1.0777x over previous
import jax
import jax.numpy as jnp
from jax import lax
from jax.experimental import pallas as pl
from jax.experimental.pallas import tpu as pltpu, tpu_sc as plsc

E = 1_600_000
D = 16
OUT_D = 46
L = 16
C = 512
NW = 32
N_CHUNKS = E // C
CPW = -(-N_CHUNKS // NW)
GRPS = C // L

_DNUMS = lax.GatherDimensionNumbers(
    offset_dims=(), collapsed_slice_dims=(0,), start_index_map=(0,))


def _lut16(table_row, idx):
    return lax.gather(table_row, idx.reshape(L, 1), _DNUMS, (1,),
                      mode=lax.GatherScatterMode.PROMISE_IN_BOUNDS)


def _sc_body(ea_hbm, tc_hbm, out_hbm,
             tc_v, in0, in1, st0, st1, sin0, sin1, sout0, sout1):
    wid = lax.axis_index("s") * 2 + lax.axis_index("c")
    pltpu.sync_copy(tc_hbm, tc_v)

    ins = (sin0, sin1)
    outs = (sout0, sout1)
    inbufs = (in0, in1)
    stages = (st0, st1)

    def in_copy(g, b):
        return pltpu.make_async_copy(
            ea_hbm.at[:, pl.ds(g * C, C)], inbufs[b], ins[b])

    def out_copy(g, b):
        return pltpu.make_async_copy(
            stages[b], out_hbm.at[:, pl.ds(g * C, C)], outs[b])

    def compute(b):
        ib = inbufs[b]
        sb = stages[b]

        @plsc.parallel_loop(0, GRPS, step=1, unroll=2)
        def grp(i):
            goff = i * L
            ti = ib[0, pl.ds(goff, L)].astype(jnp.int32)
            ri = ib[1, pl.ds(goff, L)].astype(jnp.int32)
            for j in range(D):
                sb[j, pl.ds(goff, L)] = _lut16(tc_v[j, :], ti)
            for j in range(D):
                sb[D + j, pl.ds(goff, L)] = _lut16(tc_v[D + j, :], ri)
            for c in range(2, D):
                sb[30 + c, pl.ds(goff, L)] = ib[c, pl.ds(goff, L)]

    def do_chunk(k, b):
        g = k * NW + wid

        @pl.when(g < N_CHUNKS)
        def _():
            in_copy(g, b).wait()

            @pl.when(k >= 2)
            def _():
                out_copy(g, b).wait()

            compute(b)
            out_copy(g, b).start()

            @pl.when(g + 2 * NW < N_CHUNKS)
            def _():
                in_copy(g + 2 * NW, b).start()

    @pl.when(wid < N_CHUNKS)
    def _():
        in_copy(wid, 0).start()

    @pl.when(NW + wid < N_CHUNKS)
    def _():
        in_copy(NW + wid, 1).start()

    def pair_body(i, carry):
        do_chunk(2 * i, 0)
        do_chunk(2 * i + 1, 1)
        return carry

    lax.fori_loop(0, CPW // 2, pair_body, 0)

    out_copy(wid, 0).wait()
    out_copy(wid, 1).wait()


@jax.jit
def _run(ea_t, tcols):
    mesh = plsc.VectorSubcoreMesh(core_axis_name="c", subcore_axis_name="s")
    f = pl.kernel(
        _sc_body,
        out_type=jax.ShapeDtypeStruct((OUT_D, E), jnp.float32),
        mesh=mesh,
        scratch_types=[
            pltpu.VMEM((2 * D, L), jnp.float32),
            pltpu.VMEM((D, C), jnp.float32),
            pltpu.VMEM((D, C), jnp.float32),
            pltpu.VMEM((OUT_D, C), jnp.float32),
            pltpu.VMEM((OUT_D, C), jnp.float32),
            pltpu.SemaphoreType.DMA,
            pltpu.SemaphoreType.DMA,
            pltpu.SemaphoreType.DMA,
            pltpu.SemaphoreType.DMA,
        ],
    )
    return f(ea_t, tcols)


def kernel(edge_attr, W_type, W_ring):
    tcols = jnp.concatenate([
        jnp.pad(W_type.T, ((0, 0), (0, L - 10))),
        jnp.pad(W_ring.T, ((0, 0), (0, L - 10))),
    ], axis=0)
    return _run(edge_attr.T, tcols).T
